# Initial kernel scaffold; baseline (speedup 1.0000x reference)
import jax
import jax.numpy as jnp
from jax.experimental import pallas as pl


def kernel(pred_tensor, target_tensor):
    raise NotImplementedError("write your pallas kernel here")


# speedup vs baseline: 4.8464x; 4.8464x over previous
import functools

import jax
import jax.numpy as jnp
from jax import lax
from jax.experimental import pallas as pl
from jax.experimental.pallas import tpu as pltpu
from jax.experimental.pallas import tpu_sc as plsc

S = 7
CH = 30
BATCH = 1024
M = BATCH * S * S
NC, NS, L = 2, 16, 16
NW = NC * NS
PER_W = M // NW
CHUNK = 224
N_CHUNKS = PER_W // CHUNK
GROUPS = CHUNK // L
CHUNK_F = CHUNK * CH


def _nsqrt(x):
    bits = lax.bitcast_convert_type(x, jnp.int32)
    i = jnp.int32(0x5F3759DF) - lax.shift_right_logical(bits, 1)
    y = lax.bitcast_convert_type(i, jnp.float32)
    y = y * (1.5 - 0.5 * x * y * y)
    y = y * (1.5 - 0.5 * x * y * y)
    y = y * (1.5 - 0.5 * x * y * y)
    return x * y


def _cell_group(p, t):
    t4 = t[4]
    coo = jnp.where(t4 > 0.0, 1.0, 0.0)
    noo = jnp.where(t4 == 0.0, 0.5, 0.0)

    noo_sq = (p[4] - t4) * (p[4] - t4) + (p[9] - t[9]) * (p[9] - t[9])

    inv14 = jnp.float32(1.0 / 14.0)
    t_cx = t[0] * inv14
    t_cy = t[1] * inv14
    t_ltx = t_cx - 0.5 * t[2]
    t_lty = t_cy - 0.5 * t[3]
    t_rbx = t_cx + 0.5 * t[2]
    t_rby = t_cy + 0.5 * t[3]
    a2 = t[2] * t[3]

    ious = []
    for o in (0, 5):
        cx = p[o + 0] * inv14
        cy = p[o + 1] * inv14
        ltx = jnp.maximum(cx - 0.5 * p[o + 2], t_ltx)
        lty = jnp.maximum(cy - 0.5 * p[o + 3], t_lty)
        rbx = jnp.minimum(cx + 0.5 * p[o + 2], t_rbx)
        rby = jnp.minimum(cy + 0.5 * p[o + 3], t_rby)
        w = jnp.maximum(rbx - ltx, 0.0)
        h = jnp.maximum(rby - lty, 0.0)
        inter = w * h
        a1 = p[o + 2] * p[o + 3]
        ious.append(inter / (a1 + a2 - inter))
    use1 = ious[1] > ious[0]
    max_iou = jnp.maximum(ious[0], ious[1])

    def sel(a, b):
        return jnp.where(use1, b, a)

    rx = sel(p[0], p[5])
    ry = sel(p[1], p[6])
    rw = sel(p[2], p[7])
    rh = sel(p[3], p[8])
    rconf = sel(p[4], p[9])
    nconf = sel(p[9], p[4])
    tx = sel(t[0], t[5])
    ty = sel(t[1], t[6])
    tw = sel(t[2], t[7])
    th = sel(t[3], t[8])

    dx = rx - tx
    dy = ry - ty
    dw = _nsqrt(rw) - _nsqrt(tw)
    dh = _nsqrt(rh) - _nsqrt(th)
    loc = dx * dx + dy * dy + dw * dw + dh * dh
    dc = rconf - max_iou
    contain = dc * dc

    cls = None
    for c in range(10, 30):
        d = p[c] - t[c]
        cls = d * d if cls is None else cls + d * d

    cell = coo * (5.0 * loc + 2.0 * contain + cls + nconf * nconf) + noo * noo_sq
    return cell


def _make_kernel():
    mesh = plsc.VectorSubcoreMesh(core_axis_name="c", subcore_axis_name="s")

    @functools.partial(
        pl.kernel,
        mesh=mesh,
        compiler_params=pltpu.CompilerParams(needs_layout_passes=False),
        out_type=jax.ShapeDtypeStruct((NW * L,), jnp.float32),
        scratch_types=[
            pltpu.VMEM((CHUNK_F,), jnp.float32),
            pltpu.VMEM((CHUNK_F,), jnp.float32),
            pltpu.VMEM((L,), jnp.float32),
        ],
    )
    def yolo_loss_kernel(pred_hbm, tgt_hbm, out_hbm, pbuf, tbuf, accv):
        wid = lax.axis_index("s") * NC + lax.axis_index("c")
        base_f = wid * (PER_W * CH)
        lane = lax.iota(jnp.int32, L)

        def chunk_body(ci, acc):
            off = base_f + ci * CHUNK_F
            pltpu.sync_copy(pred_hbm.at[pl.ds(off, CHUNK_F)], pbuf)
            pltpu.sync_copy(tgt_hbm.at[pl.ds(off, CHUNK_F)], tbuf)

            def group_body(g, acc_in):
                cell30 = (g * L + lane) * CH
                p = {c: plsc.load_gather(pbuf, [cell30 + c]) for c in range(CH)}
                t = {c: plsc.load_gather(tbuf, [cell30 + c]) for c in range(CH)}
                return acc_in + _cell_group(p, t)

            return lax.fori_loop(0, GROUPS, group_body, acc)

        acc = lax.fori_loop(0, N_CHUNKS, chunk_body, jnp.zeros((L,), jnp.float32))
        accv[...] = acc
        pltpu.sync_copy(accv, out_hbm.at[pl.ds(wid * L, L)])

    return yolo_loss_kernel


_KERNEL = _make_kernel()


@jax.jit
def kernel(pred_tensor, target_tensor):
    partials = _KERNEL(
        pred_tensor.reshape(M * CH),
        target_tensor.reshape(M * CH),
    )
    return jnp.sum(partials) * jnp.float32(1.0 / BATCH)


# speedup vs baseline: 8.0579x; 1.6627x over previous
import functools

import jax
import jax.numpy as jnp
from jax import lax
from jax.experimental import pallas as pl
from jax.experimental.pallas import tpu as pltpu
from jax.experimental.pallas import tpu_sc as plsc

S = 7
CH = 30
BATCH = 1024
CELLS_IMG = S * S
NC, NS, L = 2, 16, 16
NW = NC * NS
IMGS_W = BATCH // NW
CELLS_W = IMGS_W * CELLS_IMG
GROUPS = CELLS_W // L


def _nsqrt(x):
    bits = lax.bitcast_convert_type(x, jnp.int32)
    i = jnp.int32(0x5F3759DF) - lax.shift_right_logical(bits, 1)
    y = lax.bitcast_convert_type(i, jnp.float32)
    y = y * (1.5 - 0.5 * x * y * y)
    y = y * (1.5 - 0.5 * x * y * y)
    y = y * (1.5 - 0.5 * x * y * y)
    return x * y


def _cell_group(p, t):
    t4 = t[4]
    coo = jnp.where(t4 > 0.0, 1.0, 0.0)
    noo = jnp.where(t4 == 0.0, 0.5, 0.0)

    noo_sq = (p[4] - t4) * (p[4] - t4) + (p[9] - t[9]) * (p[9] - t[9])

    inv14 = jnp.float32(1.0 / 14.0)
    t_cx = t[0] * inv14
    t_cy = t[1] * inv14
    t_ltx = t_cx - 0.5 * t[2]
    t_lty = t_cy - 0.5 * t[3]
    t_rbx = t_cx + 0.5 * t[2]
    t_rby = t_cy + 0.5 * t[3]
    a2 = t[2] * t[3]

    ious = []
    for o in (0, 5):
        cx = p[o + 0] * inv14
        cy = p[o + 1] * inv14
        ltx = jnp.maximum(cx - 0.5 * p[o + 2], t_ltx)
        lty = jnp.maximum(cy - 0.5 * p[o + 3], t_lty)
        rbx = jnp.minimum(cx + 0.5 * p[o + 2], t_rbx)
        rby = jnp.minimum(cy + 0.5 * p[o + 3], t_rby)
        w = jnp.maximum(rbx - ltx, 0.0)
        h = jnp.maximum(rby - lty, 0.0)
        inter = w * h
        a1 = p[o + 2] * p[o + 3]
        ious.append(inter / (a1 + a2 - inter))
    use1 = ious[1] > ious[0]
    max_iou = jnp.maximum(ious[0], ious[1])

    def sel(a, b):
        return jnp.where(use1, b, a)

    rx = sel(p[0], p[5])
    ry = sel(p[1], p[6])
    rw = sel(p[2], p[7])
    rh = sel(p[3], p[8])
    rconf = sel(p[4], p[9])
    nconf = sel(p[9], p[4])
    tx = sel(t[0], t[5])
    ty = sel(t[1], t[6])
    tw = sel(t[2], t[7])
    th = sel(t[3], t[8])

    dx = rx - tx
    dy = ry - ty
    dw = _nsqrt(rw) - _nsqrt(tw)
    dh = _nsqrt(rh) - _nsqrt(th)
    loc = dx * dx + dy * dy + dw * dw + dh * dh
    dc = rconf - max_iou
    contain = dc * dc

    cls = None
    for c in range(10, 30):
        d = p[c] - t[c]
        cls = d * d if cls is None else cls + d * d

    cell = coo * (5.0 * loc + 2.0 * contain + cls + nconf * nconf) + noo * noo_sq
    return cell


def _make_kernel():
    mesh = plsc.VectorSubcoreMesh(core_axis_name="c", subcore_axis_name="s")

    @functools.partial(
        pl.kernel,
        mesh=mesh,
        compiler_params=pltpu.CompilerParams(needs_layout_passes=False),
        out_type=jax.ShapeDtypeStruct((NW * L,), jnp.float32),
        scratch_types=[
            pltpu.VMEM((IMGS_W, S * S * CH), jnp.float32),
            pltpu.VMEM((IMGS_W, S * S * CH), jnp.float32),
            pltpu.VMEM((L,), jnp.float32),
        ],
    )
    def yolo_loss_kernel(pred_hbm, tgt_hbm, out_hbm, pbuf, tbuf, accv):
        wid = lax.axis_index("s") * NC + lax.axis_index("c")
        img0 = wid * IMGS_W
        lane = lax.iota(jnp.int32, L)

        pltpu.sync_copy(pred_hbm.at[pl.ds(img0, IMGS_W)], pbuf)
        pltpu.sync_copy(tgt_hbm.at[pl.ds(img0, IMGS_W)], tbuf)

        def group_body(g, acc_in):
            cells = g * L + lane
            img = cells // CELLS_IMG
            rem = cells - img * CELLS_IMG
            base = rem * CH
            p = {c: plsc.load_gather(pbuf, [img, base + c]) for c in range(CH)}
            t = {c: plsc.load_gather(tbuf, [img, base + c]) for c in range(CH)}
            return acc_in + _cell_group(p, t)

        acc = lax.fori_loop(0, GROUPS, group_body, jnp.zeros((L,), jnp.float32))
        accv[...] = acc
        pltpu.sync_copy(accv, out_hbm.at[pl.ds(wid * L, L)])

    return yolo_loss_kernel


_KERNEL = _make_kernel()


@jax.jit
def kernel(pred_tensor, target_tensor):
    partials = _KERNEL(
        pred_tensor.reshape(BATCH, S * S * CH),
        target_tensor.reshape(BATCH, S * S * CH),
    )
    return jnp.sum(partials) * jnp.float32(1.0 / BATCH)


# speedup vs baseline: 8.2247x; 1.0207x over previous
import functools

import jax
import jax.numpy as jnp
from jax import lax
from jax.experimental import pallas as pl
from jax.experimental.pallas import tpu as pltpu
from jax.experimental.pallas import tpu_sc as plsc

S = 7
CH = 30
BATCH = 1024
CELLS_IMG = S * S
NC, NS, L = 2, 16, 16
NW = NC * NS
IMGS_W = BATCH // NW
CELLS_W = IMGS_W * CELLS_IMG
GROUPS = CELLS_W // L
ROWS_W = CELLS_W // 4


def _nsqrt(x):
    bits = lax.bitcast_convert_type(x, jnp.int32)
    i = jnp.int32(0x5F3759DF) - lax.shift_right_logical(bits, 1)
    y = lax.bitcast_convert_type(i, jnp.float32)
    y = y * (1.5 - 0.5 * x * y * y)
    y = y * (1.5 - 0.5 * x * y * y)
    y = y * (1.5 - 0.5 * x * y * y)
    return x * y


def _cell_group(p, t):
    t4 = t[4]
    coo = jnp.where(t4 > 0.0, 1.0, 0.0)
    noo = jnp.where(t4 == 0.0, 0.5, 0.0)

    noo_sq = (p[4] - t4) * (p[4] - t4) + (p[9] - t[9]) * (p[9] - t[9])

    inv14 = jnp.float32(1.0 / 14.0)
    t_cx = t[0] * inv14
    t_cy = t[1] * inv14
    t_ltx = t_cx - 0.5 * t[2]
    t_lty = t_cy - 0.5 * t[3]
    t_rbx = t_cx + 0.5 * t[2]
    t_rby = t_cy + 0.5 * t[3]
    a2 = t[2] * t[3]

    ious = []
    for o in (0, 5):
        cx = p[o + 0] * inv14
        cy = p[o + 1] * inv14
        ltx = jnp.maximum(cx - 0.5 * p[o + 2], t_ltx)
        lty = jnp.maximum(cy - 0.5 * p[o + 3], t_lty)
        rbx = jnp.minimum(cx + 0.5 * p[o + 2], t_rbx)
        rby = jnp.minimum(cy + 0.5 * p[o + 3], t_rby)
        w = jnp.maximum(rbx - ltx, 0.0)
        h = jnp.maximum(rby - lty, 0.0)
        inter = w * h
        a1 = p[o + 2] * p[o + 3]
        ious.append(inter / (a1 + a2 - inter))
    use1 = ious[1] > ious[0]
    max_iou = jnp.maximum(ious[0], ious[1])

    def sel(a, b):
        return jnp.where(use1, b, a)

    rx = sel(p[0], p[5])
    ry = sel(p[1], p[6])
    rw = sel(p[2], p[7])
    rh = sel(p[3], p[8])
    rconf = sel(p[4], p[9])
    nconf = sel(p[9], p[4])
    tx = sel(t[0], t[5])
    ty = sel(t[1], t[6])
    tw = sel(t[2], t[7])
    th = sel(t[3], t[8])

    dx = rx - tx
    dy = ry - ty
    sw = rw + tw - 2.0 * _nsqrt(rw * tw)
    sh = rh + th - 2.0 * _nsqrt(rh * th)
    loc = dx * dx + dy * dy + sw + sh
    dc = rconf - max_iou
    contain = dc * dc

    cls = None
    for c in range(10, 30):
        d = p[c] - t[c]
        cls = d * d if cls is None else cls + d * d

    cell = coo * (5.0 * loc + 2.0 * contain + cls + nconf * nconf) + noo * noo_sq
    return cell


def _make_kernel():
    mesh = plsc.VectorSubcoreMesh(core_axis_name="c", subcore_axis_name="s")

    @functools.partial(
        pl.kernel,
        mesh=mesh,
        compiler_params=pltpu.CompilerParams(needs_layout_passes=False),
        out_type=jax.ShapeDtypeStruct((NW * L,), jnp.float32),
        scratch_types=[
            pltpu.VMEM((IMGS_W, S * S * CH), jnp.float32),
            pltpu.VMEM((IMGS_W, S * S * CH), jnp.float32),
            pltpu.VMEM((L,), jnp.float32),
            pltpu.SemaphoreType.DMA,
            pltpu.SemaphoreType.DMA,
            pltpu.SemaphoreType.DMA,
            pltpu.SemaphoreType.DMA,
        ],
    )
    def yolo_loss_kernel(pred_hbm, tgt_hbm, out_hbm, pbuf, tbuf, accv,
                         s0, s1, s2, s3):
        wid = lax.axis_index("s") * NC + lax.axis_index("c")
        img0 = wid * IMGS_W
        lane = lax.iota(jnp.int32, L)
        half = IMGS_W // 2
        hgroups = GROUPS // 2

        cp0 = pltpu.async_copy(
            pred_hbm.at[pl.ds(img0, half)], pbuf.at[pl.ds(0, half)], s0)
        cp1 = pltpu.async_copy(
            tgt_hbm.at[pl.ds(img0, half)], tbuf.at[pl.ds(0, half)], s1)
        cp2 = pltpu.async_copy(
            pred_hbm.at[pl.ds(img0 + half, half)],
            pbuf.at[pl.ds(half, half)], s2)
        cp3 = pltpu.async_copy(
            tgt_hbm.at[pl.ds(img0 + half, half)],
            tbuf.at[pl.ds(half, half)], s3)

        def group_body(g, carry):
            acc_in, img, rem30 = carry
            p = {c: plsc.load_gather(pbuf, [img, rem30 + c]) for c in range(CH)}
            t = {c: plsc.load_gather(tbuf, [img, rem30 + c]) for c in range(CH)}
            acc_out = acc_in + _cell_group(p, t)
            rem30n = rem30 + L * CH
            wrap = rem30n >= S * S * CH
            img_n = img + jnp.where(wrap, 1, 0)
            rem30_n = rem30n - jnp.where(wrap, S * S * CH, 0)
            return acc_out, img_n, rem30_n

        zero = jnp.zeros((L,), jnp.float32)
        img_i = lane * 0
        rem30_i = lane * CH
        cp0.wait()
        cp1.wait()
        acc, img_c, rem30_c = lax.fori_loop(
            0, hgroups, group_body, (zero, img_i, rem30_i))
        cp2.wait()
        cp3.wait()
        acc, _, _ = lax.fori_loop(
            hgroups, GROUPS, group_body, (acc, img_c, rem30_c))
        accv[...] = acc
        pltpu.sync_copy(accv, out_hbm.at[pl.ds(wid * L, L)])

    return yolo_loss_kernel


_KERNEL = _make_kernel()


@jax.jit
def kernel(pred_tensor, target_tensor):
    partials = _KERNEL(
        pred_tensor.reshape(BATCH, S * S * CH),
        target_tensor.reshape(BATCH, S * S * CH),
    )
    return jnp.sum(partials) * jnp.float32(1.0 / BATCH)


# speedup vs baseline: 8.3966x; 1.0209x over previous
import functools

import jax
import jax.numpy as jnp
from jax import lax
from jax.experimental import pallas as pl
from jax.experimental.pallas import tpu as pltpu
from jax.experimental.pallas import tpu_sc as plsc

S = 7
CH = 30
BATCH = 1024
CELLS_IMG = S * S
ROW = S * S * CH
NC, NS, L = 2, 16, 16
NW = NC * NS
IMGS_W = BATCH // NW
CELLS_W = IMGS_W * CELLS_IMG
GROUPS = CELLS_W // L


def _nsqrt(x):
    bits = lax.bitcast_convert_type(x, jnp.int32)
    i = jnp.int32(0x5F3759DF) - lax.shift_right_logical(bits, 1)
    y = lax.bitcast_convert_type(i, jnp.float32)
    y = y * (1.5 - 0.5 * x * y * y)
    y = y * (1.5 - 0.5 * x * y * y)
    y = y * (1.5 - 0.5 * x * y * y)
    return x * y


def _box_group(p, t):
    t4 = t[4]
    coo = jnp.where(t4 > 0.0, 1.0, 0.0)

    inv14 = jnp.float32(1.0 / 14.0)
    t_cx = t[0] * inv14
    t_cy = t[1] * inv14
    t_ltx = t_cx - 0.5 * t[2]
    t_lty = t_cy - 0.5 * t[3]
    t_rbx = t_cx + 0.5 * t[2]
    t_rby = t_cy + 0.5 * t[3]
    a2 = t[2] * t[3]

    ious = []
    for o in (0, 5):
        cx = p[o + 0] * inv14
        cy = p[o + 1] * inv14
        ltx = jnp.maximum(cx - 0.5 * p[o + 2], t_ltx)
        lty = jnp.maximum(cy - 0.5 * p[o + 3], t_lty)
        rbx = jnp.minimum(cx + 0.5 * p[o + 2], t_rbx)
        rby = jnp.minimum(cy + 0.5 * p[o + 3], t_rby)
        w = jnp.maximum(rbx - ltx, 0.0)
        h = jnp.maximum(rby - lty, 0.0)
        inter = w * h
        a1 = p[o + 2] * p[o + 3]
        ious.append(inter / (a1 + a2 - inter))
    use1 = ious[1] > ious[0]
    max_iou = jnp.maximum(ious[0], ious[1])

    def sel(a, b):
        return jnp.where(use1, b, a)

    rx = sel(p[0], p[5])
    ry = sel(p[1], p[6])
    rw = sel(p[2], p[7])
    rh = sel(p[3], p[8])
    rconf = sel(p[4], p[9])
    nconf = sel(p[9], p[4])
    tx = sel(t[0], t[5])
    ty = sel(t[1], t[6])
    tw = sel(t[2], t[7])
    th = sel(t[3], t[8])

    dx = rx - tx
    dy = ry - ty
    sw = rw + tw - 2.0 * _nsqrt(rw * tw)
    sh = rh + th - 2.0 * _nsqrt(rh * th)
    loc = dx * dx + dy * dy + sw + sh
    dc = rconf - max_iou
    contain = dc * dc

    return coo * (5.0 * loc + 2.0 * contain + nconf * nconf)


def _make_sc_kernel():
    mesh = plsc.VectorSubcoreMesh(core_axis_name="c", subcore_axis_name="s")

    @functools.partial(
        pl.kernel,
        mesh=mesh,
        compiler_params=pltpu.CompilerParams(needs_layout_passes=False),
        out_type=jax.ShapeDtypeStruct((NW * L,), jnp.float32),
        scratch_types=[
            pltpu.VMEM((IMGS_W, ROW), jnp.float32),
            pltpu.VMEM((IMGS_W, ROW), jnp.float32),
            pltpu.VMEM((L,), jnp.float32),
            pltpu.SemaphoreType.DMA,
            pltpu.SemaphoreType.DMA,
            pltpu.SemaphoreType.DMA,
            pltpu.SemaphoreType.DMA,
        ],
    )
    def yolo_box_kernel(pred_hbm, tgt_hbm, out_hbm, pbuf, tbuf, accv,
                        s0, s1, s2, s3):
        wid = lax.axis_index("s") * NC + lax.axis_index("c")
        img0 = wid * IMGS_W
        lane = lax.iota(jnp.int32, L)
        half = IMGS_W // 2
        hgroups = GROUPS // 2

        cp0 = pltpu.async_copy(
            pred_hbm.at[pl.ds(img0, half)], pbuf.at[pl.ds(0, half)], s0)
        cp1 = pltpu.async_copy(
            tgt_hbm.at[pl.ds(img0, half)], tbuf.at[pl.ds(0, half)], s1)
        cp2 = pltpu.async_copy(
            pred_hbm.at[pl.ds(img0 + half, half)],
            pbuf.at[pl.ds(half, half)], s2)
        cp3 = pltpu.async_copy(
            tgt_hbm.at[pl.ds(img0 + half, half)],
            tbuf.at[pl.ds(half, half)], s3)

        def group_body(g, carry):
            acc_in, img, rem30 = carry
            p = {c: plsc.load_gather(pbuf, [img, rem30 + c]) for c in range(10)}
            t = {c: plsc.load_gather(tbuf, [img, rem30 + c]) for c in range(10)}
            acc_out = acc_in + _box_group(p, t)
            rem30n = rem30 + L * CH
            wrap = rem30n >= ROW
            img_n = img + jnp.where(wrap, 1, 0)
            rem30_n = rem30n - jnp.where(wrap, ROW, 0)
            return acc_out, img_n, rem30_n

        zero = jnp.zeros((L,), jnp.float32)
        img_i = lane * 0
        rem30_i = lane * CH
        cp0.wait()
        cp1.wait()
        acc, img_c, rem30_c = lax.fori_loop(
            0, hgroups, group_body, (zero, img_i, rem30_i))
        cp2.wait()
        cp3.wait()
        acc, _, _ = lax.fori_loop(
            hgroups, GROUPS, group_body, (acc, img_c, rem30_c))
        accv[...] = acc
        pltpu.sync_copy(accv, out_hbm.at[pl.ds(wid * L, L)])

    return yolo_box_kernel


_SC_KERNEL = _make_sc_kernel()

_TC_ROWS = 128


def _tc_body(p_ref, t_ref, o_ref):
    i = pl.program_id(0)

    pos = lax.broadcasted_iota(jnp.int32, (1, ROW), 1)
    chan = pos - (pos // CH) * CH
    cls_m = jnp.where(chan >= 10, 1.0, 0.0)

    rp = lax.broadcasted_iota(jnp.int32, (ROW, CELLS_IMG), 0)
    ck = lax.broadcasted_iota(jnp.int32, (ROW, CELLS_IMG), 1)
    s1 = jnp.where(rp == ck * CH + 4, 1.0, 0.0)

    rk = lax.broadcasted_iota(jnp.int32, (CELLS_IMG, ROW), 0)
    cp = lax.broadcasted_iota(jnp.int32, (CELLS_IMG, ROW), 1)
    ch2 = cp - (cp // CH) * CH
    incell = (cp // CH) == rk
    s23 = jnp.where(incell & (ch2 >= 10), -1.0, 0.0) + jnp.where(
        incell & ((ch2 == 4) | (ch2 == 9)), 0.5, 0.0)

    p = p_ref[...]
    t = t_ref[...]
    d = p - t
    sq = d * d

    t4c = lax.dot_general(t, s1, (((1,), (0,)), ((), ())),
                          preferred_element_type=jnp.float32)
    noo = jnp.where(t4c == 0.0, 1.0, 0.0)
    exp23 = lax.dot_general(noo, s23, (((1,), (0,)), ((), ())),
                            preferred_element_type=jnp.float32)

    val = jnp.sum(sq * (cls_m + exp23))

    @pl.when(i == 0)
    def _init():
        o_ref[0, 0] = 0.0

    o_ref[0, 0] += val


def _tc_part(p2, t2):
    return pl.pallas_call(
        _tc_body,
        grid=(BATCH // _TC_ROWS,),
        in_specs=[
            pl.BlockSpec((_TC_ROWS, ROW), lambda i: (i, 0)),
            pl.BlockSpec((_TC_ROWS, ROW), lambda i: (i, 0)),
        ],
        out_specs=pl.BlockSpec(memory_space=pltpu.SMEM),
        out_shape=jax.ShapeDtypeStruct((1, 1), jnp.float32),
    )(p2, t2)


@jax.jit
def kernel(pred_tensor, target_tensor):
    p2 = pred_tensor.reshape(BATCH, ROW)
    t2 = target_tensor.reshape(BATCH, ROW)
    box_partials = _SC_KERNEL(p2, t2)
    dense = _tc_part(p2, t2)
    return (jnp.sum(box_partials) + dense[0, 0]) * jnp.float32(1.0 / BATCH)


# speedup vs baseline: 8.4918x; 1.0113x over previous
import functools

import jax
import jax.numpy as jnp
from jax import lax
from jax.experimental import pallas as pl
from jax.experimental.pallas import tpu as pltpu
from jax.experimental.pallas import tpu_sc as plsc

S = 7
CH = 30
BATCH = 1024
CELLS_IMG = S * S
ROW = S * S * CH
NC, NS, L = 2, 16, 16
NW = NC * NS
IMGS_W = BATCH // NW
CELLS_W = IMGS_W * CELLS_IMG
GROUPS = CELLS_W // L


def _nsqrt(x):
    bits = lax.bitcast_convert_type(x, jnp.int32)
    i = jnp.int32(0x5F3759DF) - lax.shift_right_logical(bits, 1)
    y = lax.bitcast_convert_type(i, jnp.float32)
    y = y * (1.5 - 0.5 * x * y * y)
    y = y * (1.5 - 0.5 * x * y * y)
    y = y * (1.5 - 0.5 * x * y * y)
    return x * y


def _box_group(p, t):
    t4 = t[4]
    coo = jnp.where(t4 > 0.0, 1.0, 0.0)

    inv14 = jnp.float32(1.0 / 14.0)
    t_cx = t[0] * inv14
    t_cy = t[1] * inv14
    t_ltx = t_cx - 0.5 * t[2]
    t_lty = t_cy - 0.5 * t[3]
    t_rbx = t_cx + 0.5 * t[2]
    t_rby = t_cy + 0.5 * t[3]
    a2 = t[2] * t[3]

    ious = []
    for o in (0, 5):
        cx = p[o + 0] * inv14
        cy = p[o + 1] * inv14
        ltx = jnp.maximum(cx - 0.5 * p[o + 2], t_ltx)
        lty = jnp.maximum(cy - 0.5 * p[o + 3], t_lty)
        rbx = jnp.minimum(cx + 0.5 * p[o + 2], t_rbx)
        rby = jnp.minimum(cy + 0.5 * p[o + 3], t_rby)
        w = jnp.maximum(rbx - ltx, 0.0)
        h = jnp.maximum(rby - lty, 0.0)
        inter = w * h
        a1 = p[o + 2] * p[o + 3]
        ious.append(inter / (a1 + a2 - inter))
    use1 = ious[1] > ious[0]
    max_iou = jnp.maximum(ious[0], ious[1])

    def sel(a, b):
        return jnp.where(use1, b, a)

    rx = sel(p[0], p[5])
    ry = sel(p[1], p[6])
    rw = sel(p[2], p[7])
    rh = sel(p[3], p[8])
    rconf = sel(p[4], p[9])
    nconf = sel(p[9], p[4])
    tx = sel(t[0], t[5])
    ty = sel(t[1], t[6])
    tw = sel(t[2], t[7])
    th = sel(t[3], t[8])

    dx = rx - tx
    dy = ry - ty
    sw = rw + tw - 2.0 * _nsqrt(rw * tw)
    sh = rh + th - 2.0 * _nsqrt(rh * th)
    loc = dx * dx + dy * dy + sw + sh
    dc = rconf - max_iou
    contain = dc * dc

    return coo * (5.0 * loc + 2.0 * contain + nconf * nconf)


def _make_sc_kernel():
    mesh = plsc.VectorSubcoreMesh(core_axis_name="c", subcore_axis_name="s")

    @functools.partial(
        pl.kernel,
        mesh=mesh,
        compiler_params=pltpu.CompilerParams(needs_layout_passes=False),
        out_type=jax.ShapeDtypeStruct((NW * L,), jnp.float32),
        scratch_types=[
            pltpu.VMEM((IMGS_W, ROW), jnp.float32),
            pltpu.VMEM((IMGS_W, ROW), jnp.float32),
            pltpu.VMEM((L,), jnp.float32),
            pltpu.SemaphoreType.DMA,
            pltpu.SemaphoreType.DMA,
            pltpu.SemaphoreType.DMA,
            pltpu.SemaphoreType.DMA,
            pltpu.SemaphoreType.DMA,
            pltpu.SemaphoreType.DMA,
            pltpu.SemaphoreType.DMA,
            pltpu.SemaphoreType.DMA,
        ],
    )
    def yolo_box_kernel(pred_hbm, tgt_hbm, out_hbm, pbuf, tbuf, accv,
                        s0, s1, s2, s3, s4, s5, s6, s7):
        wid = lax.axis_index("s") * NC + lax.axis_index("c")
        img0 = wid * IMGS_W
        lane = lax.iota(jnp.int32, L)
        q = IMGS_W // 4

        sems = (s0, s1, s2, s3, s4, s5, s6, s7)
        cps = []
        for k in range(4):
            cps.append(pltpu.async_copy(
                pred_hbm.at[pl.ds(img0 + k * q, q)],
                pbuf.at[pl.ds(k * q, q)], sems[2 * k]))
            cps.append(pltpu.async_copy(
                tgt_hbm.at[pl.ds(img0 + k * q, q)],
                tbuf.at[pl.ds(k * q, q)], sems[2 * k + 1]))

        def group_body(g, carry):
            acc_in, img, rem30 = carry
            p = {c: plsc.load_gather(pbuf, [img, rem30 + c]) for c in range(10)}
            t = {c: plsc.load_gather(tbuf, [img, rem30 + c]) for c in range(10)}
            acc_out = acc_in + _box_group(p, t)
            rem30n = rem30 + L * CH
            wrap = rem30n >= ROW
            img_n = img + jnp.where(wrap, 1, 0)
            rem30_n = rem30n - jnp.where(wrap, ROW, 0)
            return acc_out, img_n, rem30_n

        carry = (jnp.zeros((L,), jnp.float32), lane * 0, lane * CH)
        cps[0].wait()
        cps[1].wait()
        carry = lax.fori_loop(0, 24, group_body, carry)
        cps[2].wait()
        cps[3].wait()
        carry = lax.fori_loop(24, 49, group_body, carry)
        cps[4].wait()
        cps[5].wait()
        carry = lax.fori_loop(49, 73, group_body, carry)
        cps[6].wait()
        cps[7].wait()
        acc, _, _ = lax.fori_loop(73, GROUPS, group_body, carry)
        accv[...] = acc
        pltpu.sync_copy(accv, out_hbm.at[pl.ds(wid * L, L)])

    return yolo_box_kernel


_SC_KERNEL = _make_sc_kernel()

_TC_ROWS = 128


def _tc_body(p_ref, t_ref, o_ref):
    i = pl.program_id(0)

    pos = lax.broadcasted_iota(jnp.int32, (1, ROW), 1)
    chan = pos - (pos // CH) * CH
    cls_m = jnp.where(chan >= 10, 1.0, 0.0)

    rp = lax.broadcasted_iota(jnp.int32, (ROW, CELLS_IMG), 0)
    ck = lax.broadcasted_iota(jnp.int32, (ROW, CELLS_IMG), 1)
    s1 = jnp.where(rp == ck * CH + 4, 1.0, 0.0)

    rk = lax.broadcasted_iota(jnp.int32, (CELLS_IMG, ROW), 0)
    cp = lax.broadcasted_iota(jnp.int32, (CELLS_IMG, ROW), 1)
    ch2 = cp - (cp // CH) * CH
    incell = (cp // CH) == rk
    s23 = jnp.where(incell & (ch2 >= 10), -1.0, 0.0) + jnp.where(
        incell & ((ch2 == 4) | (ch2 == 9)), 0.5, 0.0)

    p = p_ref[...]
    t = t_ref[...]
    d = p - t
    sq = d * d

    t4c = lax.dot_general(t, s1, (((1,), (0,)), ((), ())),
                          preferred_element_type=jnp.float32)
    noo = jnp.where(t4c == 0.0, 1.0, 0.0)
    exp23 = lax.dot_general(noo, s23, (((1,), (0,)), ((), ())),
                            preferred_element_type=jnp.float32)

    val = jnp.sum(sq * (cls_m + exp23))

    @pl.when(i == 0)
    def _init():
        o_ref[0, 0] = 0.0

    o_ref[0, 0] += val


def _tc_part(p2, t2):
    return pl.pallas_call(
        _tc_body,
        grid=(BATCH // _TC_ROWS,),
        in_specs=[
            pl.BlockSpec((_TC_ROWS, ROW), lambda i: (i, 0)),
            pl.BlockSpec((_TC_ROWS, ROW), lambda i: (i, 0)),
        ],
        out_specs=pl.BlockSpec(memory_space=pltpu.SMEM),
        out_shape=jax.ShapeDtypeStruct((1, 1), jnp.float32),
    )(p2, t2)


@jax.jit
def kernel(pred_tensor, target_tensor):
    p2 = pred_tensor.reshape(BATCH, ROW)
    t2 = target_tensor.reshape(BATCH, ROW)
    box_partials = _SC_KERNEL(p2, t2)
    dense = _tc_part(p2, t2)
    return (jnp.sum(box_partials) + dense[0, 0]) * jnp.float32(1.0 / BATCH)
